# baseline (speedup 1.0000x reference)
import jax
import jax.numpy as jnp
from jax.experimental import pallas as pl
from jax.experimental.pallas import tpu as pltpu
from jax.experimental.pallas import tpu_sc as plsc

_R = 16
_LANES = 16


def kernel(x, pos_emb):
    B, T, C = x.shape
    x2 = x.reshape(B * T, C)
    n_pe_blocks = T // _R

    mesh = plsc.VectorSubcoreMesh(core_axis_name="c", subcore_axis_name="s")

    @pl.kernel(out_type=jax.ShapeDtypeStruct((B * T, C), x.dtype), mesh=mesh)
    def sc_add(x_hbm, pe_hbm, o_hbm):
        def body(x_vmem, pe_vmem, o_vmem):
            @pl.loop(0, _R)
            def _(r):
                @pl.loop(0, C, step=_LANES)
                def _(c):
                    slc = (pl.ds(r, 1), pl.ds(c, _LANES))
                    o_vmem.at[*slc][...] = (
                        x_vmem.at[*slc][...] + pe_vmem.at[*slc][...]
                    )

        pltpu.emit_pipeline(
            body,
            grid=(B * T // _R,),
            in_specs=[
                pl.BlockSpec((_R, C), index_map=lambda i: (i, 0)),
                pl.BlockSpec((_R, C), index_map=lambda i: (i % n_pe_blocks, 0)),
            ],
            out_specs=[pl.BlockSpec((_R, C), index_map=lambda i: (i, 0))],
            core_axis_name=("c", "s"),
            dimension_semantics=(pltpu.PARALLEL,),
        )(x_hbm, pe_hbm, o_hbm)

    return sc_add(x2, pos_emb).reshape(B, T, C)


# speedup vs baseline: 1.0877x; 1.0877x over previous
import jax
import jax.numpy as jnp
from jax.experimental import pallas as pl
from jax.experimental.pallas import tpu as pltpu
from jax.experimental.pallas import tpu_sc as plsc

_R = 16
_LANES = 16


def kernel(x, pos_emb):
    B, T, C = x.shape
    x2 = x.reshape(B * T, C)
    n_pe_blocks = T // _R

    mesh = plsc.VectorSubcoreMesh(core_axis_name="c", subcore_axis_name="s")

    @pl.kernel(out_type=jax.ShapeDtypeStruct((B * T, C), x.dtype), mesh=mesh)
    def sc_add(x_hbm, pe_hbm, o_hbm):
        def body(x_vmem, pe_vmem, o_vmem):
            @pl.loop(0, _R)
            def _(r):
                for c in range(0, C, _LANES):
                    slc = (pl.ds(r, 1), pl.ds(c, _LANES))
                    o_vmem.at[*slc][...] = (
                        x_vmem.at[*slc][...] + pe_vmem.at[*slc][...]
                    )

        pltpu.emit_pipeline(
            body,
            grid=(B * T // _R,),
            in_specs=[
                pl.BlockSpec((_R, C), index_map=lambda i: (i, 0)),
                pl.BlockSpec((_R, C), index_map=lambda i: (i % n_pe_blocks, 0)),
            ],
            out_specs=[pl.BlockSpec((_R, C), index_map=lambda i: (i, 0))],
            core_axis_name=("c", "s"),
            dimension_semantics=(pltpu.PARALLEL,),
        )(x_hbm, pe_hbm, o_hbm)

    return sc_add(x2, pos_emb).reshape(B, T, C)


# speedup vs baseline: 5.0071x; 4.6033x over previous
import jax
import jax.numpy as jnp
from jax.experimental import pallas as pl
from jax.experimental.pallas import tpu as pltpu

_TB = 2048


def _add_kernel(x_ref, pe_ref, o_ref):
    o_ref[...] = x_ref[...] + pe_ref[...]


def kernel(x, pos_emb):
    B, T, C = x.shape
    grid = (T // _TB, B)
    return pl.pallas_call(
        _add_kernel,
        grid=grid,
        in_specs=[
            pl.BlockSpec((1, _TB, C), lambda t, b: (b, t, 0)),
            pl.BlockSpec((_TB, C), lambda t, b: (t, 0)),
        ],
        out_specs=pl.BlockSpec((1, _TB, C), lambda t, b: (b, t, 0)),
        out_shape=jax.ShapeDtypeStruct((B, T, C), x.dtype),
        compiler_params=pltpu.CompilerParams(
            dimension_semantics=("parallel", "arbitrary"),
        ),
    )(x, pos_emb)
